# baseline (speedup 1.0000x reference)
import functools
import math

import jax
import jax.numpy as jnp
from jax.experimental import pallas as pl
from jax.experimental.pallas import tpu as pltpu

DIM = 768
K = 32
DL = 64
S = 2048
NEG = -jnp.inf


def _rmsn(x):
    xx = x * x
    acc = xx[:, 0:128]
    for i in range(1, 6):
        acc = acc + xx[:, i * 128:(i + 1) * 128]
    w = 64
    while w >= 1:
        acc = acc[:, 0:w] + acc[:, w:2 * w]
        w //= 2
    return x / jnp.sqrt(acc / DIM + 1e-6)


def _topk32(s_row):
    idx = jax.lax.broadcasted_iota(jnp.int32, s_row.shape, 1)
    kcol = jax.lax.broadcasted_iota(jnp.int32, (K, 1), 0)
    s = s_row
    idxs = []
    vcol = jnp.zeros((K, 1), jnp.float32)
    for k in range(K):
        m = jnp.max(s)
        i0 = jnp.min(jnp.where(s == m, idx, s_row.shape[1]))
        idxs.append(i0)
        vcol = vcol + jnp.where(kcol == k, m, 0.0)
        s = jnp.where(idx == i0, NEG, s)
    return idxs, vcol


def _gather_rows(ref, idxs):
    return jnp.concatenate([ref[pl.ds(i, 1), :] for i in idxs], axis=0)


def _light_attn_body(x_ref, ctx_ref, wq_ref, wk_ref, wv_ref, wo_ref, o_ref,
                     k_scr, v_scr):
    i = pl.program_id(0)

    @pl.when(i == 0)
    def _():
        c = _rmsn(ctx_ref[...])
        k_scr[...] = c @ wk_ref[...]
        v_scr[...] = c @ wv_ref[...]

    x = x_ref[...]
    h = _rmsn(x)
    q = h @ wq_ref[...]
    s = jax.lax.dot_general(q, k_scr[...], (((1,), (1,)), ((), ()))) / jnp.sqrt(jnp.float32(DL))
    e = jnp.exp(s - jnp.max(s, axis=-1, keepdims=True))
    acc = e[:, 0:128]
    for i in range(1, S // 128):
        acc = acc + e[:, i * 128:(i + 1) * 128]
    w = 64
    while w >= 1:
        acc = acc[:, 0:w] + acc[:, w:2 * w]
        w //= 2
    p = e / acc
    o_ref[...] = x + (p @ v_scr[...]) @ wo_ref[...]


def _light_attn(x, ctx, wq, wk, wv, wo):
    nchunk = 8
    cs = S // nchunk
    return pl.pallas_call(
        _light_attn_body,
        grid=(nchunk,),
        in_specs=[
            pl.BlockSpec((cs, DIM), lambda i: (i, 0)),
            pl.BlockSpec((S, DIM), lambda i: (0, 0)),
            pl.BlockSpec((DIM, DL), lambda i: (0, 0)),
            pl.BlockSpec((DIM, DL), lambda i: (0, 0)),
            pl.BlockSpec((DIM, DL), lambda i: (0, 0)),
            pl.BlockSpec((DL, DIM), lambda i: (0, 0)),
        ],
        out_specs=pl.BlockSpec((cs, DIM), lambda i: (i, 0)),
        out_shape=jax.ShapeDtypeStruct((S, DIM), jnp.float32),
        scratch_shapes=[
            pltpu.VMEM((S, DL), jnp.float32),
            pltpu.VMEM((S, DL), jnp.float32),
        ],
    )(x, ctx, wq, wk, wv, wo)


def _heavy_attn_body(is_self, *refs):
    if is_self:
        x_ref, rq_ref, rkv_ref, wq_ref, wk_ref, wv_ref, wo_ref, o_ref = refs
        ctx_ref = x_ref
    else:
        x_ref, ctx_ref, rq_ref, rkv_ref, wq_ref, wk_ref, wv_ref, wo_ref, o_ref = refs
    x = x_ref[...]
    h = _rmsn(x)
    c = h if is_self else _rmsn(ctx_ref[...])
    sq = jax.lax.dot_general(rq_ref[...], h, (((1,), (1,)), ((), ())))
    skv = jax.lax.dot_general(rkv_ref[...], c, (((1,), (1,)), ((), ())))
    iq, vq = _topk32(sq)
    ikv, vkv = _topk32(skv)
    xq = _rmsn(_gather_rows(x_ref, iq))
    xkv = _rmsn(_gather_rows(x_ref if is_self else ctx_ref, ikv))
    q = xq @ wq_ref[...]
    k = (xkv @ wk_ref[...]) * jax.nn.sigmoid(vkv)
    v = xkv @ wv_ref[...]
    s = jax.lax.dot_general(q, k, (((1,), (1,)), ((), ()))) / jnp.sqrt(jnp.float32(DIM))
    p = jax.nn.softmax(s, axis=-1)
    o = ((p @ v) @ wo_ref[...]) * jax.nn.sigmoid(vq)
    o_ref[...] = x
    for k_i in range(K):
        row = o_ref[pl.ds(iq[k_i], 1), :]
        o_ref[pl.ds(iq[k_i], 1), :] = row + o[k_i:k_i + 1, :]


def _heavy_attn(x, ctx, rq, rkv, wq, wk, wv, wo, is_self):
    args = (x,) if is_self else (x, ctx)
    args = args + (rq.reshape(1, DIM), rkv.reshape(1, DIM), wq, wk, wv, wo)
    return pl.pallas_call(
        functools.partial(_heavy_attn_body, is_self),
        out_shape=jax.ShapeDtypeStruct((S, DIM), jnp.float32),
    )(*args)


def _lff_body(x_ref, w1_ref, w2_ref, o_ref):
    x = x_ref[...]
    h = _rmsn(x)
    o_ref[...] = x + jax.nn.gelu(h @ w1_ref[...]) @ w2_ref[...]


def _lff(x, w1, w2):
    return pl.pallas_call(
        _lff_body,
        out_shape=jax.ShapeDtypeStruct((S, DIM), jnp.float32),
    )(x, w1, w2)


def _hff_body(x_ref, r_ref, w1_ref, w2_ref, o_ref):
    x = x_ref[...]
    h = _rmsn(x)
    s = jax.lax.dot_general(r_ref[...], h, (((1,), (1,)), ((), ())))
    idxs, v = _topk32(s)
    xk = _rmsn(_gather_rows(x_ref, idxs))
    o = (jax.nn.gelu(xk @ w1_ref[...]) @ w2_ref[...]) * jax.nn.sigmoid(v)
    o_ref[...] = x
    for k_i in range(K):
        row = o_ref[pl.ds(idxs[k_i], 1), :]
        o_ref[pl.ds(idxs[k_i], 1), :] = row + o[k_i:k_i + 1, :]


def _hff(x, r, w1, w2):
    return pl.pallas_call(
        _hff_body,
        out_shape=jax.ShapeDtypeStruct((S, DIM), jnp.float32),
    )(x, r.reshape(1, DIM), w1, w2)


def _embed_body(ids_ref, *refs):
    o_ref = refs[-1]
    for j in range(8):
        o_ref[j, :] = refs[j][0, 0, :]


def _embed(table, ids):
    nrows = ids.shape[0]
    grid = (nrows // 8,)
    t3 = table.reshape(table.shape[0], 1, DIM)
    in_specs = [
        pl.BlockSpec((1, 1, DIM), functools.partial(lambda j, i, ids: (ids[i * 8 + j], 0, 0), j))
        for j in range(8)
    ]
    return pl.pallas_call(
        _embed_body,
        grid_spec=pltpu.PrefetchScalarGridSpec(
            num_scalar_prefetch=1,
            grid=grid,
            in_specs=in_specs,
            out_specs=pl.BlockSpec((8, DIM), lambda i, ids: (i, 0)),
        ),
        out_shape=jax.ShapeDtypeStruct((nrows, DIM), jnp.float32),
    )(ids, *([t3] * 8))


def _lm_head_body(y_ref, w_ref, b_ref, o_ref):
    o_ref[...] = y_ref[...] @ w_ref[...] + b_ref[...]


def _lm_head(y, w, b):
    vocab = w.shape[1]
    bw = 128
    return pl.pallas_call(
        _lm_head_body,
        grid=(vocab // bw,),
        in_specs=[
            pl.BlockSpec((S, DIM), lambda j: (0, 0)),
            pl.BlockSpec((DIM, bw), lambda j: (0, j)),
            pl.BlockSpec((1, bw), lambda j: (0, j)),
        ],
        out_specs=pl.BlockSpec((S, bw), lambda j: (0, j)),
        out_shape=jax.ShapeDtypeStruct((S, vocab), jnp.float32),
    )(y, w, b.reshape(1, vocab))


def _enc_block(x, p, pre):
    x = _light_attn(x, x, p[pre + 'la_Wq'], p[pre + 'la_Wk'], p[pre + 'la_Wv'], p[pre + 'la_Wo'])
    x = _heavy_attn(x, x, p[pre + 'ha_rq'], p[pre + 'ha_rkv'],
                    p[pre + 'ha_Wq'], p[pre + 'ha_Wk'], p[pre + 'ha_Wv'], p[pre + 'ha_Wo'], True)
    x = _lff(x, p[pre + 'lff_w1'], p[pre + 'lff_w2'])
    x = _hff(x, p[pre + 'hff_r'], p[pre + 'hff_w1'], p[pre + 'hff_w2'])
    return x


def _dec_block(x, enc, p, pre):
    x = _light_attn(x, x, p[pre + 'la_Wq'], p[pre + 'la_Wk'], p[pre + 'la_Wv'], p[pre + 'la_Wo'])
    x = _heavy_attn(x, x, p[pre + 'ha_rq'], p[pre + 'ha_rkv'],
                    p[pre + 'ha_Wq'], p[pre + 'ha_Wk'], p[pre + 'ha_Wv'], p[pre + 'ha_Wo'], True)
    x = _light_attn(x, enc, p[pre + 'lc_Wq'], p[pre + 'lc_Wk'], p[pre + 'lc_Wv'], p[pre + 'lc_Wo'])
    x = _heavy_attn(x, enc, p[pre + 'hc_rq'], p[pre + 'hc_rkv'],
                    p[pre + 'hc_Wq'], p[pre + 'hc_Wk'], p[pre + 'hc_Wv'], p[pre + 'hc_Wo'], False)
    x = _lff(x, p[pre + 'lff_w1'], p[pre + 'lff_w2'])
    x = _hff(x, p[pre + 'hff_r'], p[pre + 'hff_w1'], p[pre + 'hff_w2'])
    return x


@jax.jit
def _forward(input_ids, decoder_input_ids, params):
    p = params
    x = _embed(p['embed_enc'], input_ids.reshape(S))
    for l in range(2):
        x = _enc_block(x, p, 'enc%d_' % l)
    enc = x
    y = _embed(p['embed_dec'], decoder_input_ids.reshape(S))
    for l in range(2):
        y = _dec_block(y, enc, p, 'dec%d_' % l)
    out = _lm_head(y, p['lm_head_w'], p['lm_head_b'])
    return out.reshape(1, S, -1)


def kernel(input_ids, decoder_input_ids, params):
    return _forward(input_ids, decoder_input_ids, params)
